# Initial kernel scaffold; baseline (speedup 1.0000x reference)
import jax
import jax.numpy as jnp
from jax.experimental import pallas as pl


def kernel(contexts, lengths, targets, neg_samples, in_embed, out_embed):
    raise NotImplementedError("write your pallas kernel here")


# speedup vs baseline: 3.0994x; 3.0994x over previous
import functools

import jax
import jax.numpy as jnp
from jax import lax
from jax.experimental import pallas as pl
from jax.experimental.pallas import tpu as pltpu
from jax.experimental.pallas import tpu_sc as plsc

B, C, K, D = 16384, 20, 10, 64
NC, NS = 2, 16
NW = NC * NS
RPW = B // NW
G = 16
NG = RPW // G
CCH = (G * C) // 80
NCH = (G * K) // 80

_MESH = plsc.VectorSubcoreMesh(
    core_axis_name="c", subcore_axis_name="s", num_cores=NC, num_subcores=NS
)


def _sc_body(ctxidx_hbm, negidx_hbm, posidx_hbm, len_hbm, in_emb, out_emb,
             pos_out, neg_out,
             ctxidx_v, negidx_v, posidx_v, len_v,
             ctx_buf, neg_buf, pos_buf, pos_s, neg_s, sem):
    wid = lax.axis_index("s") * NC + lax.axis_index("c")

    pltpu.sync_copy(ctxidx_hbm.at[wid], ctxidx_v)
    pltpu.sync_copy(negidx_hbm.at[wid], negidx_v)
    pltpu.sync_copy(posidx_hbm.at[wid], posidx_v)
    pltpu.sync_copy(len_hbm.at[wid], len_v)

    iota = lax.iota(jnp.int32, 16)

    def group_body(g, carry):
        cps = []
        for i in range(CCH):
            cp = pltpu.make_async_copy(
                in_emb.at[ctxidx_v.at[g * CCH + i]],
                ctx_buf.at[pl.ds(i * 80, 80)], sem)
            cp.start()
            cps.append(cp)
        for i in range(NCH):
            cp = pltpu.make_async_copy(
                out_emb.at[negidx_v.at[g * NCH + i]],
                neg_buf.at[pl.ds(i * 80, 80)], sem)
            cp.start()
            cps.append(cp)
        cp = pltpu.make_async_copy(out_emb.at[posidx_v.at[g]], pos_buf, sem)
        cp.start()
        cps.append(cp)
        for cp in cps:
            cp.wait()

        len_f = len_v[g, :].astype(jnp.float32)
        recip = 1.0 / jnp.maximum(len_f, 1.0)

        def row_body(r, scores):
            onehot = iota == r
            accs = [ctx_buf[r * C, pl.ds(j * 16, 16)] for j in range(4)]
            for c in range(1, C):
                for j in range(4):
                    accs[j] = accs[j] + ctx_buf[r * C + c, pl.ds(j * 16, 16)]
            new_scores = []
            part = accs[0] * pos_buf[r, pl.ds(0, 16)]
            for j in range(1, 4):
                part = part + accs[j] * pos_buf[r, pl.ds(j * 16, 16)]
            new_scores.append(jnp.where(onehot, jnp.sum(part), scores[0]))
            for k in range(K):
                part = accs[0] * neg_buf[r * K + k, pl.ds(0, 16)]
                for j in range(1, 4):
                    part = part + accs[j] * neg_buf[r * K + k, pl.ds(j * 16, 16)]
                new_scores.append(
                    jnp.where(onehot, jnp.sum(part), scores[1 + k]))
            return tuple(new_scores)

        scores0 = tuple(jnp.zeros((16,), jnp.float32) for _ in range(K + 1))
        scores = lax.fori_loop(0, G, row_body, scores0)
        pos_s[pl.ds(g * G, G)] = scores[0] * recip
        for k in range(K):
            neg_s[k, pl.ds(g * G, G)] = scores[1 + k] * recip
        return carry

    lax.fori_loop(0, NG, group_body, 0)

    pltpu.sync_copy(pos_s, pos_out.at[wid])
    pltpu.sync_copy(neg_s, neg_out.at[wid])


_sc_scores = functools.partial(
    pl.kernel,
    out_type=[
        jax.ShapeDtypeStruct((NW, RPW), jnp.float32),
        jax.ShapeDtypeStruct((NW, K, RPW), jnp.float32),
    ],
    mesh=_MESH,
    compiler_params=pltpu.CompilerParams(
        needs_layout_passes=False, use_tc_tiling_on_sc=False),
    scratch_types=[
        pltpu.VMEM((NG * CCH, 80), jnp.int32),
        pltpu.VMEM((NG * NCH, 80), jnp.int32),
        pltpu.VMEM((NG, G), jnp.int32),
        pltpu.VMEM((NG, G), jnp.int32),
        pltpu.VMEM((G * C, D), jnp.float32),
        pltpu.VMEM((G * K, D), jnp.float32),
        pltpu.VMEM((G, D), jnp.float32),
        pltpu.VMEM((RPW,), jnp.float32),
        pltpu.VMEM((K, RPW), jnp.float32),
        pltpu.SemaphoreType.DMA,
    ],
)(_sc_body)


def _loss_body(pos_ref, neg_ref, out_ref):
    p = pos_ref[...]
    n = neg_ref[...]

    def logsig(x):
        return jnp.minimum(x, 0.0) - jnp.log1p(jnp.exp(-jnp.abs(x)))

    tot = jnp.sum(logsig(p)) + jnp.sum(logsig(-n))
    out_ref[0, 0] = -tot / B


_loss = pl.pallas_call(
    _loss_body,
    out_shape=jax.ShapeDtypeStruct((1, 1), jnp.float32),
    out_specs=pl.BlockSpec(memory_space=pltpu.SMEM),
)


def kernel(contexts, lengths, targets, neg_samples, in_embed, out_embed):
    ctx_idx = contexts.reshape(NW, NG * CCH, 80)
    neg_idx = neg_samples.reshape(NW, NG * NCH, 80)
    pos_idx = targets.reshape(NW, NG, G)
    len_r = lengths.reshape(NW, NG, G)
    pos_sc, neg_sc = _sc_scores(ctx_idx, neg_idx, pos_idx, len_r,
                                in_embed, out_embed)
    loss = _loss(pos_sc.reshape(128, 128), neg_sc.reshape(1280, 128))
    return loss[0, 0]
